# baseline (speedup 1.0000x reference)
import functools

import numpy as np

import jax
import jax.numpy as jnp
from jax import lax
from jax.experimental import pallas as pl
from jax.experimental.pallas import tpu as pltpu
from jax.experimental.pallas import tpu_sc as plsc

N_NODES = 100000
N_EDGES = 6400000
NC = 2
NS = 16
L = 16
NW = NC * NS

SL = 6272
N_PAD = NS * SL
ECH = 8000
NCHG = N_EDGES // ECH
CPW = NCHG // NW

LI = np.int32(L)
SLI = np.int32(SL)
ECHI = np.int32(ECH)
CPWI = np.int32(CPW)
NCI = np.int32(NC)
I1 = np.int32(1)

_mesh = plsc.VectorSubcoreMesh(
    core_axis_name="c", subcore_axis_name="s", num_cores=NC, num_subcores=NS)
_params = pltpu.CompilerParams(needs_layout_passes=False)


def _sloop(length, body):
    def _step(i, _):
        body(i)
        return i + np.int32(1), None

    lax.scan(_step, np.int32(0), None, length=length)


@functools.partial(
    pl.kernel,
    out_type=jax.ShapeDtypeStruct((2, NC, NS, N_PAD), jnp.float32),
    mesh=_mesh,
    compiler_params=_params,
    scratch_types=[
        pltpu.VMEM((N_PAD,), jnp.float32),
        pltpu.VMEM((2 * ECH,), jnp.int32),
        pltpu.VMEM((ECH,), jnp.float32),
    ],
)
def _degree_kernel(ei_hbm, w_hbm, out_hbm, tab, ixv, wv):
    c = lax.axis_index("c")
    s = lax.axis_index("s")
    wid = s * NCI + c
    t0 = wid * CPWI
    zero16 = jnp.zeros((L,), jnp.float32)
    iota2 = lax.iota(jnp.int32, L) * np.int32(2)

    for p in (np.int32(0), np.int32(1)):
        def _z(i):
            tab[pl.ds(i * LI, L)] = zero16

        _sloop(N_PAD // L, _z)

        def _chunk(i):
            e0 = (t0 + i) * ECHI
            pltpu.sync_copy(ei_hbm.at[p, pl.ds(e0 * np.int32(2), 2 * ECH)],
                            ixv)
            pltpu.sync_copy(w_hbm.at[pl.ds(e0, ECH)], wv)

            def _acc(k):
                i16 = plsc.load_gather(ixv, [k * np.int32(2 * L) + iota2])
                plsc.addupdate_scatter(tab, [i16], wv[pl.ds(k * LI, L)])

            _sloop(ECH // L, _acc)

        _sloop(CPW, _chunk)
        pltpu.sync_copy(tab, out_hbm.at[p, c, s])


@functools.partial(
    pl.kernel,
    out_type=jax.ShapeDtypeStruct((N_EDGES,), jnp.float32),
    mesh=_mesh,
    compiler_params=_params,
    scratch_types=[
        pltpu.VMEM_SHARED((N_PAD,), jnp.float32),
        pltpu.VMEM((N_PAD,), jnp.float32),
        pltpu.VMEM((2 * ECH,), jnp.int32),
        pltpu.VMEM((ECH,), jnp.float32),
    ],
)
def _apply_kernel(ei_hbm, w_hbm, part_hbm, out_hbm, norm_sh, tab, ixv, av):
    c = lax.axis_index("c")
    s = lax.axis_index("s")
    wid = s * NCI + c
    t0 = wid * CPWI
    iota2 = lax.iota(jnp.int32, L) * np.int32(2)

    for p, val_hbm in ((np.int32(0), w_hbm), (np.int32(1), out_hbm)):
        acc = tab.at[pl.ds(0, SL)]
        tmp = tab.at[pl.ds(SL, SL)]
        first = True
        for cc in range(NC):
            for k in range(NS):
                dstbuf = acc if first else tmp
                pltpu.sync_copy(
                    part_hbm.at[p, np.int32(cc), np.int32(k),
                                pl.ds(s * SLI, SL)], dstbuf)
                if not first:
                    def _add(i):
                        sl = pl.ds(i * LI, L)
                        acc[sl] = acc[sl] + tmp[sl]

                    _sloop(SL // L, _add)
                first = False

        def _nrm(i):
            sl = pl.ds(i * LI, L)
            x = acc[sl]
            xi = plsc.bitcast(x, jnp.int32)
            yi = np.int32(0x5F3759DF) - (xi >> 1)
            y = plsc.bitcast(yi, jnp.float32)
            y = y * (1.5 - 0.5 * x * y * y)
            y = y * (1.5 - 0.5 * x * y * y)
            y = y * (1.5 - 0.5 * x * y * y)
            acc[sl] = y

        _sloop(SL // L, _nrm)

        pltpu.sync_copy(acc, norm_sh.at[pl.ds(s * SLI, SL)])
        plsc.subcore_barrier()
        pltpu.sync_copy(norm_sh, tab)
        plsc.subcore_barrier()

        def _chunk(i):
            e0 = (t0 + i) * ECHI
            pltpu.sync_copy(ei_hbm.at[p, pl.ds(e0 * np.int32(2), 2 * ECH)],
                            ixv)
            pltpu.sync_copy(val_hbm.at[pl.ds(e0, ECH)], av)

            def _app(k):
                sl = pl.ds(k * LI, L)
                i16 = plsc.load_gather(ixv, [k * np.int32(2 * L) + iota2])
                g = plsc.load_gather(tab, [i16])
                av[sl] = av[sl] * g

            _sloop(ECH // L, _app)
            pltpu.sync_copy(av, out_hbm.at[pl.ds(e0, ECH)])

        _sloop(CPW, _chunk)


def kernel(edge_index, edge_weight):
    ei = jax.lax.bitcast_convert_type(edge_index, jnp.int32)
    ei = ei.reshape(2, 2 * N_EDGES)
    w = edge_weight.astype(jnp.float32)
    parts = _degree_kernel(ei, w)
    return _apply_kernel(ei, w, parts)


# speedup vs baseline: 18.7990x; 18.7990x over previous
import functools

import numpy as np

import jax
import jax.numpy as jnp
from jax import lax
from jax.experimental import pallas as pl
from jax.experimental.pallas import tpu as pltpu
from jax.experimental.pallas import tpu_sc as plsc

N_NODES = 100000
N_EDGES = 6400000
NC = 2
NS = 16
L = 16
NW = NC * NS

SL = 6272
N_PAD = NS * SL
ECH = 8000
NCHG = N_EDGES // ECH
CPW = NCHG // NW

LI = np.int32(L)
SLI = np.int32(SL)
ECHI = np.int32(ECH)
CPWI = np.int32(CPW)
NCI = np.int32(NC)
I1 = np.int32(1)

_mesh = plsc.VectorSubcoreMesh(
    core_axis_name="c", subcore_axis_name="s", num_cores=NC, num_subcores=NS)
_params = pltpu.CompilerParams(needs_layout_passes=False)


def _sloop(length, body):
    def _step(i, _):
        body(i)
        return i + np.int32(1), None

    lax.scan(_step, np.int32(0), None, length=length)


@functools.partial(
    pl.kernel,
    out_type=jax.ShapeDtypeStruct((2, NC, NS, N_PAD), jnp.float32),
    mesh=_mesh,
    compiler_params=_params,
    scratch_types=[
        pltpu.VMEM((N_PAD,), jnp.float32),
        pltpu.VMEM((ECH,), jnp.int32),
        pltpu.VMEM((ECH,), jnp.float32),
    ],
)
def _degree_kernel(src_hbm, dst_hbm, w_hbm, out_hbm, tab, ixv, wv):
    c = lax.axis_index("c")
    s = lax.axis_index("s")
    wid = s * NCI + c
    t0 = wid * CPWI
    zero16 = jnp.zeros((L,), jnp.float32)

    for p, idx_hbm in ((np.int32(0), src_hbm), (np.int32(1), dst_hbm)):
        def _z(i):
            tab[pl.ds(i * LI, L)] = zero16

        _sloop(N_PAD // L, _z)

        def _chunk(i):
            e0 = (t0 + i) * ECHI
            pltpu.sync_copy(idx_hbm.at[pl.ds(e0, ECH)], ixv)
            pltpu.sync_copy(w_hbm.at[pl.ds(e0, ECH)], wv)

            def _acc(k):
                sl = pl.ds(k * LI, L)
                plsc.addupdate_scatter(tab, [ixv[sl]], wv[sl])

            _sloop(ECH // L, _acc)

        _sloop(CPW, _chunk)
        pltpu.sync_copy(tab, out_hbm.at[p, c, s])


@functools.partial(
    pl.kernel,
    out_type=jax.ShapeDtypeStruct((N_EDGES,), jnp.float32),
    mesh=_mesh,
    compiler_params=_params,
    scratch_types=[
        pltpu.VMEM_SHARED((N_PAD,), jnp.float32),
        pltpu.VMEM((N_PAD,), jnp.float32),
        pltpu.VMEM((ECH,), jnp.int32),
        pltpu.VMEM((ECH,), jnp.float32),
    ],
)
def _apply_kernel(src_hbm, dst_hbm, w_hbm, part_hbm, out_hbm,
                  norm_sh, tab, ixv, av):
    c = lax.axis_index("c")
    s = lax.axis_index("s")
    wid = s * NCI + c
    t0 = wid * CPWI

    for p, idx_hbm, val_hbm in ((np.int32(0), src_hbm, w_hbm),
                                (np.int32(1), dst_hbm, out_hbm)):
        acc = tab.at[pl.ds(0, SL)]
        tmp = tab.at[pl.ds(SL, SL)]
        first = True
        for cc in range(NC):
            for k in range(NS):
                dstbuf = acc if first else tmp
                pltpu.sync_copy(
                    part_hbm.at[p, np.int32(cc), np.int32(k),
                                pl.ds(s * SLI, SL)], dstbuf)
                if not first:
                    def _add(i):
                        sl = pl.ds(i * LI, L)
                        acc[sl] = acc[sl] + tmp[sl]

                    _sloop(SL // L, _add)
                first = False

        def _nrm(i):
            sl = pl.ds(i * LI, L)
            x = acc[sl]
            xi = plsc.bitcast(x, jnp.int32)
            yi = np.int32(0x5F3759DF) - (xi >> 1)
            y = plsc.bitcast(yi, jnp.float32)
            y = y * (1.5 - 0.5 * x * y * y)
            y = y * (1.5 - 0.5 * x * y * y)
            y = y * (1.5 - 0.5 * x * y * y)
            acc[sl] = y

        _sloop(SL // L, _nrm)

        pltpu.sync_copy(acc, norm_sh.at[pl.ds(s * SLI, SL)])
        plsc.subcore_barrier()
        pltpu.sync_copy(norm_sh, tab)
        plsc.subcore_barrier()

        def _chunk(i):
            e0 = (t0 + i) * ECHI
            pltpu.sync_copy(idx_hbm.at[pl.ds(e0, ECH)], ixv)
            pltpu.sync_copy(val_hbm.at[pl.ds(e0, ECH)], av)

            def _app(k):
                sl = pl.ds(k * LI, L)
                g = plsc.load_gather(tab, [ixv[sl]])
                av[sl] = av[sl] * g

            _sloop(ECH // L, _app)
            pltpu.sync_copy(av, out_hbm.at[pl.ds(e0, ECH)])

        _sloop(CPW, _chunk)


def kernel(edge_index, edge_weight):
    ei = edge_index.astype(jnp.int32)
    src = ei[0]
    dst = ei[1]
    w = edge_weight.astype(jnp.float32)
    parts = _degree_kernel(src, dst, w)
    return _apply_kernel(src, dst, w, parts)


# speedup vs baseline: 22.8333x; 1.2146x over previous
import functools

import numpy as np

import jax
import jax.numpy as jnp
from jax import lax
from jax.experimental import pallas as pl
from jax.experimental.pallas import tpu as pltpu
from jax.experimental.pallas import tpu_sc as plsc

N_NODES = 100000
N_EDGES = 6400000
NC = 2
NS = 16
L = 16
NW = NC * NS

SL = 6272
N_PAD = NS * SL
ECH = 4000
NCHG = N_EDGES // ECH
CPW = NCHG // NW

LI = np.int32(L)
SLI = np.int32(SL)
ECHI = np.int32(ECH)
CPWI = np.int32(CPW)
NCI = np.int32(NC)
I1 = np.int32(1)

_mesh = plsc.VectorSubcoreMesh(
    core_axis_name="c", subcore_axis_name="s", num_cores=NC, num_subcores=NS)
_params = pltpu.CompilerParams(needs_layout_passes=False)


def _sloop(length, body):
    def _step(i, _):
        body(i)
        return i + np.int32(1), None

    lax.scan(_step, np.int32(0), None, length=length)


@functools.partial(
    pl.kernel,
    out_type=jax.ShapeDtypeStruct((2, NC, NS, N_PAD), jnp.float32),
    mesh=_mesh,
    compiler_params=_params,
    scratch_types=[
        pltpu.VMEM((N_PAD,), jnp.float32),
        pltpu.VMEM((ECH,), jnp.int32),
        pltpu.VMEM((ECH,), jnp.int32),
        pltpu.VMEM((ECH,), jnp.float32),
        pltpu.VMEM((ECH,), jnp.float32),
        pltpu.SemaphoreType.DMA,
        pltpu.SemaphoreType.DMA,
        pltpu.SemaphoreType.DMA,
        pltpu.SemaphoreType.DMA,
    ],
)
def _degree_kernel(src_hbm, dst_hbm, w_hbm, out_hbm, tab,
                   ixv0, ixv1, wv0, wv1, si0, si1, sv0, sv1):
    c = lax.axis_index("c")
    s = lax.axis_index("s")
    wid = s * NCI + c
    t0 = wid * CPWI
    zero16 = jnp.zeros((L,), jnp.float32)
    ixb = (ixv0, ixv1)
    wvb = (wv0, wv1)
    sib = (si0, si1)
    svb = (sv0, sv1)

    for p, idx_hbm in ((np.int32(0), src_hbm), (np.int32(1), dst_hbm)):
        def _z(i):
            tab[pl.ds(i * LI, L)] = zero16

        _sloop(N_PAD // L, _z)

        def _start(j):
            q = j % 2
            e0 = (t0 + np.int32(j)) * ECHI
            d1 = pltpu.async_copy(idx_hbm.at[pl.ds(e0, ECH)], ixb[q], sib[q])
            d2 = pltpu.async_copy(w_hbm.at[pl.ds(e0, ECH)], wvb[q], svb[q])
            return d1, d2

        cur = _start(0)
        for i in range(CPW):
            q = i % 2
            nxt = _start(i + 1) if i + 1 < CPW else None
            cur[0].wait()
            cur[1].wait()
            ixv = ixb[q]
            wv = wvb[q]

            def _acc(k):
                sl = pl.ds(k * LI, L)
                plsc.addupdate_scatter(tab, [ixv[sl]], wv[sl])

            _sloop(ECH // L, _acc)
            cur = nxt

        pltpu.sync_copy(tab, out_hbm.at[p, c, s])


@functools.partial(
    pl.kernel,
    out_type=jax.ShapeDtypeStruct((N_EDGES,), jnp.float32),
    mesh=_mesh,
    compiler_params=_params,
    scratch_types=[
        pltpu.VMEM_SHARED((N_PAD,), jnp.float32),
        pltpu.VMEM((N_PAD,), jnp.float32),
        pltpu.VMEM((ECH,), jnp.int32),
        pltpu.VMEM((ECH,), jnp.int32),
        pltpu.VMEM((ECH,), jnp.float32),
        pltpu.VMEM((ECH,), jnp.float32),
        pltpu.SemaphoreType.DMA,
        pltpu.SemaphoreType.DMA,
        pltpu.SemaphoreType.DMA,
        pltpu.SemaphoreType.DMA,
        pltpu.SemaphoreType.DMA,
        pltpu.SemaphoreType.DMA,
    ],
)
def _apply_kernel(src_hbm, dst_hbm, w_hbm, part_hbm, out_hbm, norm_sh, tab,
                  ixv0, ixv1, av0, av1, si0, si1, sv0, sv1, so0, so1):
    c = lax.axis_index("c")
    s = lax.axis_index("s")
    wid = s * NCI + c
    t0 = wid * CPWI
    ixb = (ixv0, ixv1)
    avb = (av0, av1)
    sib = (si0, si1)
    svb = (sv0, sv1)
    sob = (so0, so1)

    for p, idx_hbm, val_hbm in ((np.int32(0), src_hbm, w_hbm),
                                (np.int32(1), dst_hbm, out_hbm)):
        acc = tab.at[pl.ds(0, SL)]
        tmps = (tab.at[pl.ds(SL, SL)], tab.at[pl.ds(2 * SL, SL)])
        srcs = [(np.int32(cc), np.int32(k))
                for cc in range(NC) for k in range(NS)]

        def _rsrc(j):
            cc, k = srcs[j]
            return part_hbm.at[p, cc, k, pl.ds(s * SLI, SL)]

        pltpu.sync_copy(_rsrc(0), acc)
        rcur = pltpu.async_copy(_rsrc(1), tmps[0], sib[0])
        for j in range(1, NC * NS):
            q = (j - 1) % 2
            rnxt = (pltpu.async_copy(_rsrc(j + 1), tmps[1 - q], sib[1 - q])
                    if j + 1 < NC * NS else None)
            rcur.wait()
            tmp = tmps[q]

            def _add(i):
                sl = pl.ds(i * LI, L)
                acc[sl] = acc[sl] + tmp[sl]

            _sloop(SL // L, _add)
            rcur = rnxt

        def _nrm(i):
            sl = pl.ds(i * LI, L)
            x = acc[sl]
            xi = plsc.bitcast(x, jnp.int32)
            yi = np.int32(0x5F3759DF) - (xi >> 1)
            y = plsc.bitcast(yi, jnp.float32)
            y = y * (1.5 - 0.5 * x * y * y)
            y = y * (1.5 - 0.5 * x * y * y)
            y = y * (1.5 - 0.5 * x * y * y)
            acc[sl] = y

        _sloop(SL // L, _nrm)

        pltpu.sync_copy(acc, norm_sh.at[pl.ds(s * SLI, SL)])
        plsc.subcore_barrier()
        pltpu.sync_copy(norm_sh, tab)
        plsc.subcore_barrier()

        out_desc = [None, None]

        def _start(j):
            q = j % 2
            if out_desc[q] is not None:
                out_desc[q].wait()
                out_desc[q] = None
            e0 = (t0 + np.int32(j)) * ECHI
            d1 = pltpu.async_copy(idx_hbm.at[pl.ds(e0, ECH)], ixb[q], sib[q])
            d2 = pltpu.async_copy(val_hbm.at[pl.ds(e0, ECH)], avb[q], svb[q])
            return d1, d2

        cur = _start(0)
        for i in range(CPW):
            q = i % 2
            nxt = _start(i + 1) if i + 1 < CPW else None
            cur[0].wait()
            cur[1].wait()
            ixv = ixb[q]
            av = avb[q]

            def _app(k):
                sl = pl.ds(k * LI, L)
                g = plsc.load_gather(tab, [ixv[sl]])
                av[sl] = av[sl] * g

            _sloop(ECH // L, _app)
            e0 = (t0 + np.int32(i)) * ECHI
            out_desc[q] = pltpu.async_copy(av, out_hbm.at[pl.ds(e0, ECH)],
                                           sob[q])
            cur = nxt

        for q in (0, 1):
            if out_desc[q] is not None:
                out_desc[q].wait()


def kernel(edge_index, edge_weight):
    ei = edge_index.astype(jnp.int32)
    src = ei[0]
    dst = ei[1]
    w = edge_weight.astype(jnp.float32)
    parts = _degree_kernel(src, dst, w)
    return _apply_kernel(src, dst, w, parts)


# speedup vs baseline: 24.1486x; 1.0576x over previous
import functools

import numpy as np

import jax
import jax.numpy as jnp
from jax import lax
from jax.experimental import pallas as pl
from jax.experimental.pallas import tpu as pltpu
from jax.experimental.pallas import tpu_sc as plsc

N_NODES = 100000
N_EDGES = 6400000
NC = 2
NS = 16
L = 16
NW = NC * NS

SL = 6272
N_PAD = NS * SL
ECH = 4000
NCHG = N_EDGES // ECH
CPW = NCHG // NW

LI = np.int32(L)
SLI = np.int32(SL)
ECHI = np.int32(ECH)
CPWI = np.int32(CPW)
NCI = np.int32(NC)
I1 = np.int32(1)

_mesh = plsc.VectorSubcoreMesh(
    core_axis_name="c", subcore_axis_name="s", num_cores=NC, num_subcores=NS)
_params = pltpu.CompilerParams(needs_layout_passes=False)


def _sloop(length, body, unroll=1):
    def _step(i, _):
        body(i)
        return i + np.int32(1), None

    lax.scan(_step, np.int32(0), None, length=length, unroll=unroll)


@functools.partial(
    pl.kernel,
    out_type=jax.ShapeDtypeStruct((NC, NS, N_PAD), jnp.float32),
    mesh=_mesh,
    compiler_params=_params,
    scratch_types=[
        pltpu.VMEM((N_PAD,), jnp.float32),
        pltpu.VMEM((ECH,), jnp.int32),
        pltpu.VMEM((ECH,), jnp.int32),
        pltpu.VMEM((ECH,), jnp.float32),
        pltpu.VMEM((ECH,), jnp.float32),
        pltpu.SemaphoreType.DMA,
        pltpu.SemaphoreType.DMA,
        pltpu.SemaphoreType.DMA,
        pltpu.SemaphoreType.DMA,
    ],
)
def _degree_kernel(idx_hbm, w_hbm, out_hbm, tab,
                   ixv0, ixv1, wv0, wv1, si0, si1, sv0, sv1):
    c = lax.axis_index("c")
    s = lax.axis_index("s")
    wid = s * NCI + c
    t0 = wid * CPWI
    zero16 = jnp.zeros((L,), jnp.float32)
    ixb = (ixv0, ixv1)
    wvb = (wv0, wv1)
    sib = (si0, si1)
    svb = (sv0, sv1)

    def _z(i):
        tab[pl.ds(i * LI, L)] = zero16

    _sloop(N_PAD // L, _z, unroll=8)

    def _start(j):
        q = j % 2
        e0 = (t0 + np.int32(j)) * ECHI
        d1 = pltpu.async_copy(idx_hbm.at[pl.ds(e0, ECH)], ixb[q], sib[q])
        d2 = pltpu.async_copy(w_hbm.at[pl.ds(e0, ECH)], wvb[q], svb[q])
        return d1, d2

    cur = _start(0)
    for i in range(CPW):
        q = i % 2
        nxt = _start(i + 1) if i + 1 < CPW else None
        cur[0].wait()
        cur[1].wait()
        ixv = ixb[q]
        wv = wvb[q]

        def _acc(k):
            sl = pl.ds(k * LI, L)
            plsc.addupdate_scatter(tab, [ixv[sl]], wv[sl])

        _sloop(ECH // L, _acc, unroll=5)
        cur = nxt

    pltpu.sync_copy(tab, out_hbm.at[c, s])


@functools.partial(
    pl.kernel,
    out_type=jax.ShapeDtypeStruct((N_EDGES,), jnp.float32),
    mesh=_mesh,
    compiler_params=_params,
    scratch_types=[
        pltpu.VMEM_SHARED((N_PAD,), jnp.float32),
        pltpu.VMEM((N_PAD,), jnp.float32),
        pltpu.VMEM((ECH,), jnp.int32),
        pltpu.VMEM((ECH,), jnp.int32),
        pltpu.VMEM((ECH,), jnp.float32),
        pltpu.VMEM((ECH,), jnp.float32),
        pltpu.SemaphoreType.DMA,
        pltpu.SemaphoreType.DMA,
        pltpu.SemaphoreType.DMA,
        pltpu.SemaphoreType.DMA,
        pltpu.SemaphoreType.DMA,
        pltpu.SemaphoreType.DMA,
    ],
)
def _apply_kernel(src_hbm, dst_hbm, w_hbm, psrc_hbm, pdst_hbm, out_hbm,
                  norm_sh, tab,
                  ixv0, ixv1, av0, av1, si0, si1, sv0, sv1, so0, so1):
    c = lax.axis_index("c")
    s = lax.axis_index("s")
    wid = s * NCI + c
    t0 = wid * CPWI
    ixb = (ixv0, ixv1)
    avb = (av0, av1)
    sib = (si0, si1)
    svb = (sv0, sv1)
    sob = (so0, so1)

    for part_hbm, idx_hbm, val_hbm in ((psrc_hbm, src_hbm, w_hbm),
                                       (pdst_hbm, dst_hbm, out_hbm)):
        acc = tab.at[pl.ds(0, SL)]
        tmps = (tab.at[pl.ds(SL, SL)], tab.at[pl.ds(2 * SL, SL)])
        srcs = [(np.int32(cc), np.int32(k))
                for cc in range(NC) for k in range(NS)]

        def _rsrc(j):
            cc, k = srcs[j]
            return part_hbm.at[cc, k, pl.ds(s * SLI, SL)]

        pltpu.sync_copy(_rsrc(0), acc)
        rcur = pltpu.async_copy(_rsrc(1), tmps[0], sib[0])
        for j in range(1, NC * NS):
            q = (j - 1) % 2
            rnxt = (pltpu.async_copy(_rsrc(j + 1), tmps[1 - q], sib[1 - q])
                    if j + 1 < NC * NS else None)
            rcur.wait()
            tmp = tmps[q]

            def _add(i):
                sl = pl.ds(i * LI, L)
                acc[sl] = acc[sl] + tmp[sl]

            _sloop(SL // L, _add, unroll=8)
            rcur = rnxt

        def _nrm(i):
            sl = pl.ds(i * LI, L)
            x = acc[sl]
            xi = plsc.bitcast(x, jnp.int32)
            yi = np.int32(0x5F3759DF) - (xi >> 1)
            y = plsc.bitcast(yi, jnp.float32)
            y = y * (1.5 - 0.5 * x * y * y)
            y = y * (1.5 - 0.5 * x * y * y)
            y = y * (1.5 - 0.5 * x * y * y)
            acc[sl] = y

        _sloop(SL // L, _nrm, unroll=4)

        pltpu.sync_copy(acc, norm_sh.at[pl.ds(s * SLI, SL)])
        plsc.subcore_barrier()
        pltpu.sync_copy(norm_sh, tab)
        plsc.subcore_barrier()

        out_desc = [None, None]

        def _start(j):
            q = j % 2
            if out_desc[q] is not None:
                out_desc[q].wait()
                out_desc[q] = None
            e0 = (t0 + np.int32(j)) * ECHI
            d1 = pltpu.async_copy(idx_hbm.at[pl.ds(e0, ECH)], ixb[q], sib[q])
            d2 = pltpu.async_copy(val_hbm.at[pl.ds(e0, ECH)], avb[q], svb[q])
            return d1, d2

        cur = _start(0)
        for i in range(CPW):
            q = i % 2
            nxt = _start(i + 1) if i + 1 < CPW else None
            cur[0].wait()
            cur[1].wait()
            ixv = ixb[q]
            av = avb[q]

            def _app(k):
                sl = pl.ds(k * LI, L)
                g = plsc.load_gather(tab, [ixv[sl]])
                av[sl] = av[sl] * g

            _sloop(ECH // L, _app, unroll=5)
            e0 = (t0 + np.int32(i)) * ECHI
            out_desc[q] = pltpu.async_copy(av, out_hbm.at[pl.ds(e0, ECH)],
                                           sob[q])
            cur = nxt

        for q in (0, 1):
            if out_desc[q] is not None:
                out_desc[q].wait()


def kernel(edge_index, edge_weight):
    src = edge_index[0].astype(jnp.int32)
    dst = edge_index[1].astype(jnp.int32)
    w = edge_weight.astype(jnp.float32)
    psrc = _degree_kernel(src, w)
    pdst = _degree_kernel(dst, w)
    return _apply_kernel(src, dst, w, psrc, pdst)


# speedup vs baseline: 26.7567x; 1.1080x over previous
import functools

import numpy as np

import jax
import jax.numpy as jnp
from jax import lax
from jax.experimental import pallas as pl
from jax.experimental.pallas import tpu as pltpu
from jax.experimental.pallas import tpu_sc as plsc

N_NODES = 100000
N_EDGES = 6400000
NC = 2
NS = 16
L = 16
NW = NC * NS

SL = 6272
N_PAD = NS * SL
ECH = 4000
NCHG = N_EDGES // ECH
CPW = NCHG // NW
ECHA = 2000
NCHGA = N_EDGES // ECHA
CPWA = NCHGA // NW

LI = np.int32(L)
SLI = np.int32(SL)
ECHI = np.int32(ECH)
ECHAI = np.int32(ECHA)
CPWAI = np.int32(CPWA)
CPWI = np.int32(CPW)
NCI = np.int32(NC)
I1 = np.int32(1)

_mesh = plsc.VectorSubcoreMesh(
    core_axis_name="c", subcore_axis_name="s", num_cores=NC, num_subcores=NS)
_params = pltpu.CompilerParams(needs_layout_passes=False)


def _sloop(length, body, unroll=1):
    def _step(i, _):
        body(i)
        return i + np.int32(1), None

    lax.scan(_step, np.int32(0), None, length=length, unroll=unroll)


@functools.partial(
    pl.kernel,
    out_type=jax.ShapeDtypeStruct((NC, NS, N_PAD), jnp.float32),
    mesh=_mesh,
    compiler_params=_params,
    scratch_types=[
        pltpu.VMEM((N_PAD,), jnp.float32),
        pltpu.VMEM((ECH,), jnp.int32),
        pltpu.VMEM((ECH,), jnp.int32),
        pltpu.VMEM((ECH,), jnp.float32),
        pltpu.VMEM((ECH,), jnp.float32),
        pltpu.SemaphoreType.DMA,
        pltpu.SemaphoreType.DMA,
        pltpu.SemaphoreType.DMA,
        pltpu.SemaphoreType.DMA,
    ],
)
def _degree_kernel(idx_hbm, w_hbm, out_hbm, tab,
                   ixv0, ixv1, wv0, wv1, si0, si1, sv0, sv1):
    c = lax.axis_index("c")
    s = lax.axis_index("s")
    wid = s * NCI + c
    t0 = wid * CPWI
    zero16 = jnp.zeros((L,), jnp.float32)
    ixb = (ixv0, ixv1)
    wvb = (wv0, wv1)
    sib = (si0, si1)
    svb = (sv0, sv1)

    def _z(i):
        tab[pl.ds(i * LI, L)] = zero16

    _sloop(N_PAD // L, _z, unroll=8)

    def _start(j):
        q = j % 2
        e0 = (t0 + np.int32(j)) * ECHI
        d1 = pltpu.async_copy(idx_hbm.at[pl.ds(e0, ECH)], ixb[q], sib[q])
        d2 = pltpu.async_copy(w_hbm.at[pl.ds(e0, ECH)], wvb[q], svb[q])
        return d1, d2

    cur = _start(0)
    for i in range(CPW):
        q = i % 2
        nxt = _start(i + 1) if i + 1 < CPW else None
        cur[0].wait()
        cur[1].wait()
        ixv = ixb[q]
        wv = wvb[q]

        def _acc(k):
            sl = pl.ds(k * LI, L)
            plsc.addupdate_scatter(tab, [ixv[sl]], wv[sl])

        _sloop(ECH // L, _acc, unroll=5)
        cur = nxt

    pltpu.sync_copy(tab, out_hbm.at[c, s])


@functools.partial(
    pl.kernel,
    out_type=jax.ShapeDtypeStruct((N_EDGES,), jnp.float32),
    mesh=_mesh,
    compiler_params=_params,
    scratch_types=[
        pltpu.VMEM_SHARED((N_PAD,), jnp.float32),
        pltpu.VMEM((N_PAD,), jnp.float32),
        pltpu.VMEM((ECHA,), jnp.int32),
        pltpu.VMEM((ECHA,), jnp.int32),
        pltpu.VMEM((ECHA,), jnp.int32),
        pltpu.VMEM((ECHA,), jnp.int32),
        pltpu.VMEM((ECHA,), jnp.float32),
        pltpu.VMEM((ECHA,), jnp.float32),
        pltpu.SemaphoreType.DMA,
        pltpu.SemaphoreType.DMA,
        pltpu.SemaphoreType.DMA,
        pltpu.SemaphoreType.DMA,
        pltpu.SemaphoreType.DMA,
        pltpu.SemaphoreType.DMA,
        pltpu.SemaphoreType.DMA,
        pltpu.SemaphoreType.DMA,
    ],
)
def _apply_kernel(src_hbm, dst_hbm, w_hbm, psrc_hbm, pdst_hbm, out_hbm,
                  norm_sh, tab, ixs0, ixs1, ixd0, ixd1, av0, av1,
                  ss0, ss1, sd0, sd1, sv0, sv1, so0, so1):
    c = lax.axis_index("c")
    s = lax.axis_index("s")
    wid = s * NCI + c
    t0 = wid * CPWAI
    isb = (ixs0, ixs1)
    idb = (ixd0, ixd1)
    avb = (av0, av1)
    ssb = (ss0, ss1)
    sdb = (sd0, sd1)
    svb = (sv0, sv1)
    sob = (so0, so1)

    acc_s = tab.at[pl.ds(0, SL)]
    acc_d = tab.at[pl.ds(SL, SL)]
    tmps = (tab.at[pl.ds(2 * SL, SL)], tab.at[pl.ds(3 * SL, SL)])
    srcs = [(np.int32(cc), np.int32(k))
            for cc in range(NC) for k in range(NS)]

    for part_hbm, acc in ((psrc_hbm, acc_s), (pdst_hbm, acc_d)):
        def _rsrc(j):
            cc, k = srcs[j]
            return part_hbm.at[cc, k, pl.ds(s * SLI, SL)]

        pltpu.sync_copy(_rsrc(0), acc)
        rcur = pltpu.async_copy(_rsrc(1), tmps[0], ssb[0])
        for j in range(1, NC * NS):
            q = (j - 1) % 2
            rnxt = (pltpu.async_copy(_rsrc(j + 1), tmps[1 - q], ssb[1 - q])
                    if j + 1 < NC * NS else None)
            rcur.wait()
            tmp = tmps[q]

            def _add(i):
                sl = pl.ds(i * LI, L)
                acc[sl] = acc[sl] + tmp[sl]

            _sloop(SL // L, _add, unroll=8)
            rcur = rnxt

        def _nrm(i):
            sl = pl.ds(i * LI, L)
            x = acc[sl]
            xi = plsc.bitcast(x, jnp.int32)
            yi = np.int32(0x5F3759DF) - (xi >> 1)
            y = plsc.bitcast(yi, jnp.float32)
            y = y * (1.5 - 0.5 * x * y * y)
            y = y * (1.5 - 0.5 * x * y * y)
            y = y * (1.5 - 0.5 * x * y * y)
            acc[sl] = y

        _sloop(SL // L, _nrm, unroll=4)

    def _pk(i):
        sl = pl.ds(i * LI, L)
        pk = plsc.pack(acc_s[sl], acc_d[sl], format=plsc.PackFormat.INTERLEAVED)
        acc_s[sl] = plsc.bitcast(pk, jnp.float32)

    _sloop(SL // L, _pk, unroll=4)

    pltpu.sync_copy(acc_s, norm_sh.at[pl.ds(s * SLI, SL)])
    plsc.subcore_barrier()
    pltpu.sync_copy(norm_sh, tab)
    plsc.subcore_barrier()

    def _in_descs(q, e0):
        return (
            pltpu.make_async_copy(src_hbm.at[pl.ds(e0, ECHA)], isb[q], ssb[q]),
            pltpu.make_async_copy(dst_hbm.at[pl.ds(e0, ECHA)], idb[q], sdb[q]),
            pltpu.make_async_copy(w_hbm.at[pl.ds(e0, ECHA)], avb[q], svb[q]),
        )

    def _issue_in(q, e0):
        pltpu.async_copy(src_hbm.at[pl.ds(e0, ECHA)], isb[q], ssb[q])
        pltpu.async_copy(dst_hbm.at[pl.ds(e0, ECHA)], idb[q], sdb[q])
        pltpu.async_copy(w_hbm.at[pl.ds(e0, ECHA)], avb[q], svb[q])

    _issue_in(0, t0 * ECHAI)

    def _parity_body(q, i):
        e0 = (t0 + i) * ECHAI
        for d in _in_descs(q, e0):
            d.wait()

        @pl.when(i + I1 < CPWAI)
        def _():
            @pl.when(i >= I1)
            def _():
                ep = (t0 + i - I1) * ECHAI
                pltpu.make_async_copy(
                    avb[1 - q], out_hbm.at[pl.ds(ep, ECHA)],
                    sob[1 - q]).wait()

            _issue_in(1 - q, (t0 + i + I1) * ECHAI)

        ixs = isb[q]
        ixd = idb[q]
        av = avb[q]

        def _app(k):
            sl = pl.ds(k * LI, L)
            g1 = plsc.load_gather(tab, [ixs[sl]])
            g2 = plsc.load_gather(tab, [ixd[sl]])
            sn, _u1 = plsc.unpack(plsc.bitcast(g1, jnp.bfloat16),
                                  format=plsc.PackFormat.INTERLEAVED)
            _u2, dn = plsc.unpack(plsc.bitcast(g2, jnp.bfloat16),
                                  format=plsc.PackFormat.INTERLEAVED)
            av[sl] = av[sl] * sn * dn

        _sloop(ECHA // L, _app, unroll=5)
        pltpu.async_copy(av, out_hbm.at[pl.ds(e0, ECHA)], sob[q])

    def _chunk(i):
        @pl.when((i & I1) == np.int32(0))
        def _():
            _parity_body(0, i)

        @pl.when((i & I1) == I1)
        def _():
            _parity_body(1, i)

    _sloop(CPWA, _chunk)

    for j in (CPWA - 2, CPWA - 1):
        q = j % 2
        e0 = (t0 + np.int32(j)) * ECHAI
        pltpu.make_async_copy(avb[q], out_hbm.at[pl.ds(e0, ECHA)],
                              sob[q]).wait()


def kernel(edge_index, edge_weight):
    src = edge_index[0].astype(jnp.int32)
    dst64, _ = lax.optimization_barrier((edge_index[1], src))
    dst = dst64.astype(jnp.int32)
    w = edge_weight.astype(jnp.float32)
    psrc = _degree_kernel(src, w)
    pdst = _degree_kernel(dst, w)
    return _apply_kernel(src, dst, w, psrc, pdst)
